# baseline (speedup 1.0000x reference)
import functools

import jax
import jax.numpy as jnp
from jax.experimental import pallas as pl

_S = 64.0
_M = 0.35

_BLOCK_ROWS = 16


def _scale_body(lab_ref, x_ref, o_ref):
    x = x_ref[...]
    lab = lab_ref[...]
    col = jax.lax.broadcasted_iota(jnp.int32, x.shape, 1)
    delta = jnp.where(col == lab, -_M * _S, 0.0).astype(x.dtype)
    o_ref[...] = x * _S + delta


@jax.jit
def kernel(logits, labels):
    rows, cols = logits.shape
    lab2d = labels.astype(jnp.int32).reshape(rows, 1)
    return pl.pallas_call(
        _scale_body,
        grid=(rows // _BLOCK_ROWS,),
        in_specs=[
            pl.BlockSpec((_BLOCK_ROWS, 1), lambda j: (j, 0)),
            pl.BlockSpec((_BLOCK_ROWS, cols), lambda j: (j, 0)),
        ],
        out_specs=pl.BlockSpec((_BLOCK_ROWS, cols), lambda j: (j, 0)),
        out_shape=jax.ShapeDtypeStruct((rows, cols), logits.dtype),
    )(lab2d, logits)
